# baseline (speedup 1.0000x reference)
import functools

import jax
import jax.numpy as jnp
from jax import lax
from jax.experimental import pallas as pl
from jax.experimental.pallas import tpu as pltpu
from jax.experimental.pallas import tpu_sc as plsc

NC = 2
NS = 16
NW = NC * NS


NB = 3


def _make_sc_scatter(n_pad, h, nch0, nch1, chunk):
    rpt = n_pad // NS

    mesh = plsc.VectorSubcoreMesh(core_axis_name="c", subcore_axis_name="s")

    @functools.partial(
        pl.kernel,
        out_type=jax.ShapeDtypeStruct((NC, n_pad, h), jnp.float32),
        mesh=mesh,
        scratch_types=[
            pltpu.VMEM((NB, chunk), jnp.int32),
            pltpu.VMEM((NB, chunk), jnp.int32),
            pltpu.VMEM((NB, chunk, h), jnp.float32),
            pltpu.VMEM_SHARED((n_pad, h), jnp.float32),
            [pltpu.SemaphoreType.DMA] * NB,
            [pltpu.SemaphoreType.DMA] * NB,
            [pltpu.SemaphoreType.DMA] * NB,
        ],
    )
    def sc_scatter(h_hbm, src_hbm, dst_hbm, zero_hbm, out_hbm,
                   srcv, dstv, rows, acc_sh, sem_i, sem_g, sem_s):
        c = lax.axis_index("c")
        s = lax.axis_index("s")

        pltpu.sync_copy(zero_hbm.at[pl.ds(s * rpt, rpt)],
                        acc_sh.at[pl.ds(s * rpt, rpt)])
        plsc.subcore_barrier()

        def run_pipe(nchunks, base):
            def stage_i(j, b):
                off = pl.multiple_of(base + j * chunk, 8)
                pltpu.async_copy(src_hbm.at[pl.ds(off, chunk)], srcv.at[b],
                                 sem_i[b])
                pltpu.async_copy(dst_hbm.at[pl.ds(off, chunk)], dstv.at[b],
                                 sem_i[b])

            def stage_a(j, b):
                off = pl.multiple_of(base + j * chunk, 8)
                pltpu.make_async_copy(src_hbm.at[pl.ds(off, chunk)],
                                      srcv.at[b], sem_i[b]).wait()
                pltpu.make_async_copy(dst_hbm.at[pl.ds(off, chunk)],
                                      dstv.at[b], sem_i[b]).wait()
                pltpu.async_copy(h_hbm.at[srcv.at[b]], rows.at[b], sem_g[b])

            def stage_b(j, b):
                pltpu.make_async_copy(h_hbm.at[srcv.at[b]], rows.at[b],
                                      sem_g[b]).wait()
                pltpu.async_copy(rows.at[b], acc_sh.at[dstv.at[b]], sem_s[b],
                                 add=True)

            def stage_c(j, b):
                pltpu.make_async_copy(rows.at[b], acc_sh.at[dstv.at[b]],
                                      sem_s[b]).wait()

            stage_i(0, 0)
            stage_i(1, 1)
            stage_a(0, 0)
            stage_i(2, 2)
            stage_a(1, 1)
            stage_b(0, 0)

            @pl.loop(1, nchunks - 2, step=NB)
            def _steady(i2):
                for bb in range(NB):
                    j = i2 + bb
                    stage_c(j - 1, bb)
                    stage_i(j + 2, bb)
                    stage_a(j + 1, (bb + 2) % NB)
                    stage_b(j, (bb + 1) % NB)

            stage_c(nchunks - 3, (nchunks - 3) % NB)
            stage_a(nchunks - 1, (nchunks - 1) % NB)
            stage_b(nchunks - 2, (nchunks - 2) % NB)
            stage_c(nchunks - 2, (nchunks - 2) % NB)
            stage_b(nchunks - 1, (nchunks - 1) % NB)
            stage_c(nchunks - 1, (nchunks - 1) % NB)

        @pl.when(c == 0)
        def _core0():
            run_pipe(nch0, s * nch0 * chunk)

        @pl.when(c == 1)
        def _core1():
            run_pipe(nch1, (NS * nch0 + s * nch1) * chunk)

        plsc.subcore_barrier()
        pltpu.sync_copy(acc_sh.at[pl.ds(s * rpt, rpt)],
                        out_hbm.at[c, pl.ds(s * rpt, rpt)])

    return sc_scatter


def _make_sc_count(n_pad, nchunks, chunk):
    rpt = n_pad // NS
    rptz = ((rpt + 15) // 16) * 16
    epw = nchunks * chunk

    mesh = plsc.VectorSubcoreMesh(core_axis_name="c", subcore_axis_name="s")

    @functools.partial(
        pl.kernel,
        out_type=jax.ShapeDtypeStruct((NC * 2 * n_pad,), jnp.float32),
        mesh=mesh,
        scratch_types=[
            pltpu.VMEM((NB, chunk), jnp.int32),
            pltpu.VMEM((NB, chunk), jnp.int32),
            pltpu.VMEM((NB, chunk), jnp.float32),
            pltpu.VMEM((chunk,), jnp.float32),
            pltpu.VMEM((rptz,), jnp.float32),
            pltpu.VMEM_SHARED((n_pad,), jnp.float32),
            pltpu.VMEM_SHARED((n_pad,), jnp.float32),
            [pltpu.SemaphoreType.DMA] * NB,
            [pltpu.SemaphoreType.DMA] * NB,
            [pltpu.SemaphoreType.DMA] * NB,
        ],
    )
    def sc_count(xf_hbm, src_hbm, dst_hbm, out_hbm,
                 srcv, dstv, vals, ones, zb, acc_d, acc_x,
                 sem_i, sem_g, sem_s):
        c = lax.axis_index("c")
        s = lax.axis_index("s")
        wid = s * NC + c
        base = wid * epw

        for k in range(chunk // 16):
            ones[pl.ds(k * 16, 16)] = jnp.full((16,), 1.0, jnp.float32)
        for k in range(rptz // 16):
            zb[pl.ds(k * 16, 16)] = jnp.zeros((16,), jnp.float32)
        pltpu.sync_copy(zb.at[pl.ds(0, rpt)], acc_d.at[pl.ds(s * rpt, rpt)])
        pltpu.sync_copy(zb.at[pl.ds(0, rpt)], acc_x.at[pl.ds(s * rpt, rpt)])
        plsc.subcore_barrier()

        def stage_i(j, b):
            off = pl.multiple_of(base + j * chunk, 8)
            pltpu.async_copy(src_hbm.at[pl.ds(off, chunk)], srcv.at[b],
                             sem_i[b])
            pltpu.async_copy(dst_hbm.at[pl.ds(off, chunk)], dstv.at[b],
                             sem_i[b])

        def stage_a(j, b):
            off = pl.multiple_of(base + j * chunk, 8)
            pltpu.make_async_copy(src_hbm.at[pl.ds(off, chunk)], srcv.at[b],
                                  sem_i[b]).wait()
            pltpu.make_async_copy(dst_hbm.at[pl.ds(off, chunk)], dstv.at[b],
                                  sem_i[b]).wait()
            pltpu.async_copy(xf_hbm.at[srcv.at[b]], vals.at[b], sem_g[b])

        def stage_b(j, b):
            pltpu.make_async_copy(xf_hbm.at[srcv.at[b]], vals.at[b],
                                  sem_g[b]).wait()
            pltpu.async_copy(vals.at[b], acc_x.at[dstv.at[b]], sem_s[b],
                             add=True)
            pltpu.async_copy(ones, acc_d.at[dstv.at[b]], sem_s[b], add=True)

        def stage_c(j, b):
            pltpu.make_async_copy(vals.at[b], acc_x.at[dstv.at[b]],
                                  sem_s[b]).wait()
            pltpu.make_async_copy(ones, acc_d.at[dstv.at[b]],
                                  sem_s[b]).wait()

        stage_i(0, 0)
        stage_i(1, 1)
        stage_a(0, 0)
        stage_i(2, 2)
        stage_a(1, 1)
        stage_b(0, 0)

        @pl.loop(1, nchunks - 2, step=NB)
        def _steady(i2):
            for bb in range(NB):
                j = i2 + bb
                stage_c(j - 1, bb)
                stage_i(j + 2, bb)
                stage_a(j + 1, (bb + 2) % NB)
                stage_b(j, (bb + 1) % NB)

        stage_c(nchunks - 3, (nchunks - 3) % NB)
        stage_a(nchunks - 1, (nchunks - 1) % NB)
        stage_b(nchunks - 2, (nchunks - 2) % NB)
        stage_c(nchunks - 2, (nchunks - 2) % NB)
        stage_b(nchunks - 1, (nchunks - 1) % NB)
        stage_c(nchunks - 1, (nchunks - 1) % NB)

        plsc.subcore_barrier()
        pltpu.sync_copy(acc_d.at[pl.ds(s * rpt, rpt)], zb.at[pl.ds(0, rpt)])
        pltpu.sync_copy(zb.at[pl.ds(0, rpt)],
                        out_hbm.at[pl.ds(c * 2 * n_pad + s * rpt, rpt)])
        pltpu.sync_copy(acc_x.at[pl.ds(s * rpt, rpt)], zb.at[pl.ds(0, rpt)])
        pltpu.sync_copy(zb.at[pl.ds(0, rpt)],
                        out_hbm.at[pl.ds((c * 2 + 1) * n_pad + s * rpt,
                                         rpt)])

    return sc_count


def _embed(xf, emb_ref):
    return emb_ref[0:1, :] * (1.0 - xf) + emb_ref[1:2, :] * xf


def _bn(z, g, b):
    mu = jnp.mean(z, axis=0, keepdims=True)
    zc = z - mu
    var = jnp.mean(zc * zc, axis=0, keepdims=True)
    return zc * lax.rsqrt(var + 1e-5) * g + b


def _gin_tail(z, w1_ref, w2_ref, g1_ref, b1_ref, g2_ref, b2_ref, out_ref):
    z = jnp.dot(z, w1_ref[...], preferred_element_type=jnp.float32)
    z = _bn(z, g1_ref[...], b1_ref[...])
    z = jnp.maximum(z, 0.0)
    z = jnp.dot(z, w2_ref[...], preferred_element_type=jnp.float32)
    z = _bn(z, g2_ref[...], b2_ref[...])
    out_ref[...] = jnp.where(z > 0.0, z, jnp.exp(jnp.minimum(z, 0.0)) - 1.0)


def _layer_body(h_ref, agg_ref, w1_ref, w2_ref, g1_ref, b1_ref, g2_ref,
                b2_ref, out_ref):
    n = h_ref.shape[0]
    z = h_ref[...] + agg_ref[0, :n, :] + agg_ref[1, :n, :]
    _gin_tail(z, w1_ref, w2_ref, g1_ref, b1_ref, g2_ref, b2_ref, out_ref)


def _layer1_body(xf_ref, emb_ref, d0_ref, x0_ref, d1_ref, x1_ref, w1_ref,
                 w2_ref, g1_ref, b1_ref, g2_ref, b2_ref, out_ref):
    xf = xf_ref[...]
    e0 = emb_ref[0:1, :]
    e1 = emb_ref[1:2, :]
    h0 = e0 * (1.0 - xf) + e1 * xf
    deg = d0_ref[...] + d1_ref[...]
    s1 = x0_ref[...] + x1_ref[...]
    z = h0 + (deg - s1) * e0 + s1 * e1
    _gin_tail(z, w1_ref, w2_ref, g1_ref, b1_ref, g2_ref, b2_ref, out_ref)


def _readout_body(xf_ref, emb_ref, h1_ref, h2_ref, h3_ref, wr1_ref, br1_ref,
                  wr2_ref, br2_ref, out_ref):
    hdim = h1_ref.shape[1]
    h0 = _embed(xf_ref[...], emb_ref)
    acc = jnp.dot(h0, wr1_ref[0 * hdim:1 * hdim, :],
                  preferred_element_type=jnp.float32)
    acc += jnp.dot(h1_ref[...], wr1_ref[1 * hdim:2 * hdim, :],
                   preferred_element_type=jnp.float32)
    acc += jnp.dot(h2_ref[...], wr1_ref[2 * hdim:3 * hdim, :],
                   preferred_element_type=jnp.float32)
    acc += jnp.dot(h3_ref[...], wr1_ref[3 * hdim:4 * hdim, :],
                   preferred_element_type=jnp.float32)
    acc = jnp.maximum(acc + br1_ref[...], 0.0)
    out_ref[...] = jnp.dot(acc, wr2_ref[...],
                           preferred_element_type=jnp.float32) + br2_ref[...]


def _tc_call(body, out_shape, *args):
    return pl.pallas_call(
        body, out_shape=jax.ShapeDtypeStruct(out_shape, jnp.float32))(*args)


def kernel(x, edge_index, emb, W1, W2, bn1_g, bn1_b, bn2_g, bn2_b,
           Wr1, br1, Wr2, br2):
    n = x.shape[0]
    e = edge_index.shape[1]
    hdim = emb.shape[1]
    nlayers = W1.shape[0]

    chunk = 128
    T = -(-e // (NS * chunk))
    T = ((T + 5) // 6) * 6
    e_pad = T * NS * chunk
    nchunks_cnt = T // 2
    nch0 = max(6, min(T - 6, int(round(0.85 * T / 3.0)) * 3))
    nch1 = T - nch0
    n_pad = ((n + 8 * NS - 1) // (8 * NS)) * (8 * NS)
    if e_pad > e and n_pad == n:
        n_pad += 8 * NS

    src = edge_index[0].astype(jnp.int32)
    dst = edge_index[1].astype(jnp.int32)
    if e_pad > e:
        src = jnp.concatenate([src, jnp.zeros((e_pad - e,), jnp.int32)])
        dst = jnp.concatenate([dst, jnp.full((e_pad - e,), n, jnp.int32)])
    xf = x.astype(jnp.float32).reshape(n, 1)
    zero = jnp.zeros((n_pad, hdim), jnp.float32)
    xf1 = x.astype(jnp.float32)

    sc_scatter = _make_sc_scatter(n_pad, hdim, nch0, nch1, chunk)
    sc_count = _make_sc_count(n_pad, nchunks_cnt, chunk)

    def bn_args(i):
        return (bn1_g[i].reshape(1, hdim), bn1_b[i].reshape(1, hdim),
                bn2_g[i].reshape(1, hdim), bn2_b[i].reshape(1, hdim))

    cnt4 = sc_count(xf1, src, dst).reshape(4, n_pad)
    cnt_args = [cnt4[k, :n].reshape(n, 1) for k in range(4)]
    h = _tc_call(_layer1_body, (n, hdim), xf, emb, *cnt_args, W1[0], W2[0],
                 *bn_args(0))
    hs = [h]
    for i in range(1, nlayers):
        agg = sc_scatter(h, src, dst, zero)
        h = _tc_call(_layer_body, (n, hdim), h, agg, W1[i], W2[i],
                     *bn_args(i))
        hs.append(h)

    out = _tc_call(_readout_body, (n, 1), xf, emb, hs[0], hs[1], hs[2],
                   Wr1, br1.reshape(1, hdim), Wr2, br2.reshape(1, 1))
    return out


# speedup vs baseline: 2.3078x; 2.3078x over previous
import functools

import jax
import jax.numpy as jnp
from jax import lax
from jax.experimental import pallas as pl
from jax.experimental.pallas import tpu as pltpu
from jax.experimental.pallas import tpu_sc as plsc

NC = 2
NS = 16
NW = NC * NS


NB = 3


def _make_sc_scatter(n_pad, h, nch0, nch1, chunk):
    rpt = n_pad // NS

    mesh = plsc.VectorSubcoreMesh(core_axis_name="c", subcore_axis_name="s")

    @functools.partial(
        pl.kernel,
        out_type=jax.ShapeDtypeStruct((NC, n_pad, h), jnp.float32),
        mesh=mesh,
        scratch_types=[
            pltpu.VMEM((NB, chunk), jnp.int32),
            pltpu.VMEM((NB, chunk), jnp.int32),
            pltpu.VMEM((NB, chunk, h), jnp.float32),
            pltpu.VMEM_SHARED((n_pad, h), jnp.float32),
            [pltpu.SemaphoreType.DMA] * NB,
            [pltpu.SemaphoreType.DMA] * NB,
            [pltpu.SemaphoreType.DMA] * NB,
        ],
    )
    def sc_scatter(h_hbm, src_hbm, dst_hbm, zero_hbm, out_hbm,
                   srcv, dstv, rows, acc_sh, sem_i, sem_g, sem_s):
        c = lax.axis_index("c")
        s = lax.axis_index("s")

        pltpu.sync_copy(zero_hbm.at[pl.ds(s * rpt, rpt)],
                        acc_sh.at[pl.ds(s * rpt, rpt)])
        plsc.subcore_barrier()

        def run_pipe(nchunks, base):
            def stage_i(j, b):
                off = pl.multiple_of(base + j * chunk, 8)
                pltpu.async_copy(src_hbm.at[pl.ds(off, chunk)], srcv.at[b],
                                 sem_i[b])
                pltpu.async_copy(dst_hbm.at[pl.ds(off, chunk)], dstv.at[b],
                                 sem_i[b])

            def stage_a(j, b):
                off = pl.multiple_of(base + j * chunk, 8)
                pltpu.make_async_copy(src_hbm.at[pl.ds(off, chunk)],
                                      srcv.at[b], sem_i[b]).wait()
                pltpu.make_async_copy(dst_hbm.at[pl.ds(off, chunk)],
                                      dstv.at[b], sem_i[b]).wait()
                pltpu.async_copy(h_hbm.at[srcv.at[b]], rows.at[b], sem_g[b])

            def stage_b(j, b):
                pltpu.make_async_copy(h_hbm.at[srcv.at[b]], rows.at[b],
                                      sem_g[b]).wait()
                pltpu.async_copy(rows.at[b], acc_sh.at[dstv.at[b]], sem_s[b],
                                 add=True)

            def stage_c(j, b):
                pltpu.make_async_copy(rows.at[b], acc_sh.at[dstv.at[b]],
                                      sem_s[b]).wait()

            stage_i(0, 0)
            stage_i(1, 1)
            stage_a(0, 0)
            stage_i(2, 2)
            stage_a(1, 1)
            stage_b(0, 0)

            @pl.loop(1, nchunks - 2, step=NB)
            def _steady(i2):
                for bb in range(NB):
                    j = i2 + bb
                    stage_c(j - 1, bb)
                    stage_i(j + 2, bb)
                    stage_a(j + 1, (bb + 2) % NB)
                    stage_b(j, (bb + 1) % NB)

            stage_c(nchunks - 3, (nchunks - 3) % NB)
            stage_a(nchunks - 1, (nchunks - 1) % NB)
            stage_b(nchunks - 2, (nchunks - 2) % NB)
            stage_c(nchunks - 2, (nchunks - 2) % NB)
            stage_b(nchunks - 1, (nchunks - 1) % NB)
            stage_c(nchunks - 1, (nchunks - 1) % NB)

        @pl.when(c == 0)
        def _core0():
            run_pipe(nch0, s * nch0 * chunk)

        @pl.when(c == 1)
        def _core1():
            run_pipe(nch1, (NS * nch0 + s * nch1) * chunk)

        plsc.subcore_barrier()
        pltpu.sync_copy(acc_sh.at[pl.ds(s * rpt, rpt)],
                        out_hbm.at[c, pl.ds(s * rpt, rpt)])

    return sc_scatter


def _make_sc_count(n_pad, nchunks, chunk):
    rpt = n_pad // NS
    rptz = ((rpt + 15) // 16) * 16
    epw = nchunks * chunk

    mesh = plsc.VectorSubcoreMesh(core_axis_name="c", subcore_axis_name="s")

    @functools.partial(
        pl.kernel,
        out_type=jax.ShapeDtypeStruct((NC * 2 * n_pad,), jnp.float32),
        mesh=mesh,
        scratch_types=[
            pltpu.VMEM((NB, chunk), jnp.int32),
            pltpu.VMEM((NB, chunk), jnp.int32),
            pltpu.VMEM((NB, chunk), jnp.float32),
            pltpu.VMEM((chunk,), jnp.float32),
            pltpu.VMEM((rptz,), jnp.float32),
            pltpu.VMEM_SHARED((n_pad,), jnp.float32),
            pltpu.VMEM_SHARED((n_pad,), jnp.float32),
            [pltpu.SemaphoreType.DMA] * NB,
            [pltpu.SemaphoreType.DMA] * NB,
            [pltpu.SemaphoreType.DMA] * NB,
        ],
    )
    def sc_count(xf_hbm, src_hbm, dst_hbm, out_hbm,
                 srcv, dstv, vals, ones, zb, acc_d, acc_x,
                 sem_i, sem_g, sem_s):
        c = lax.axis_index("c")
        s = lax.axis_index("s")
        wid = s * NC + c
        base = wid * epw

        for k in range(chunk // 16):
            ones[pl.ds(k * 16, 16)] = jnp.full((16,), 1.0, jnp.float32)
        for k in range(rptz // 16):
            zb[pl.ds(k * 16, 16)] = jnp.zeros((16,), jnp.float32)
        pltpu.sync_copy(zb.at[pl.ds(0, rpt)], acc_d.at[pl.ds(s * rpt, rpt)])
        pltpu.sync_copy(zb.at[pl.ds(0, rpt)], acc_x.at[pl.ds(s * rpt, rpt)])
        plsc.subcore_barrier()

        def stage_i(j, b):
            off = pl.multiple_of(base + j * chunk, 8)
            pltpu.async_copy(src_hbm.at[pl.ds(off, chunk)], srcv.at[b],
                             sem_i[b])
            pltpu.async_copy(dst_hbm.at[pl.ds(off, chunk)], dstv.at[b],
                             sem_i[b])

        def stage_a(j, b):
            off = pl.multiple_of(base + j * chunk, 8)
            pltpu.make_async_copy(src_hbm.at[pl.ds(off, chunk)], srcv.at[b],
                                  sem_i[b]).wait()
            pltpu.make_async_copy(dst_hbm.at[pl.ds(off, chunk)], dstv.at[b],
                                  sem_i[b]).wait()
            pltpu.async_copy(xf_hbm.at[srcv.at[b]], vals.at[b], sem_g[b])

        def stage_b(j, b):
            pltpu.make_async_copy(xf_hbm.at[srcv.at[b]], vals.at[b],
                                  sem_g[b]).wait()
            pltpu.async_copy(vals.at[b], acc_x.at[dstv.at[b]], sem_s[b],
                             add=True)
            pltpu.async_copy(ones, acc_d.at[dstv.at[b]], sem_s[b], add=True)

        def stage_c(j, b):
            pltpu.make_async_copy(vals.at[b], acc_x.at[dstv.at[b]],
                                  sem_s[b]).wait()
            pltpu.make_async_copy(ones, acc_d.at[dstv.at[b]],
                                  sem_s[b]).wait()

        stage_i(0, 0)
        stage_i(1, 1)
        stage_a(0, 0)
        stage_i(2, 2)
        stage_a(1, 1)
        stage_b(0, 0)

        @pl.loop(1, nchunks - 2, step=NB)
        def _steady(i2):
            for bb in range(NB):
                j = i2 + bb
                stage_c(j - 1, bb)
                stage_i(j + 2, bb)
                stage_a(j + 1, (bb + 2) % NB)
                stage_b(j, (bb + 1) % NB)

        stage_c(nchunks - 3, (nchunks - 3) % NB)
        stage_a(nchunks - 1, (nchunks - 1) % NB)
        stage_b(nchunks - 2, (nchunks - 2) % NB)
        stage_c(nchunks - 2, (nchunks - 2) % NB)
        stage_b(nchunks - 1, (nchunks - 1) % NB)
        stage_c(nchunks - 1, (nchunks - 1) % NB)

        plsc.subcore_barrier()
        pltpu.sync_copy(acc_d.at[pl.ds(s * rpt, rpt)], zb.at[pl.ds(0, rpt)])
        pltpu.sync_copy(zb.at[pl.ds(0, rpt)],
                        out_hbm.at[pl.ds(c * 2 * n_pad + s * rpt, rpt)])
        pltpu.sync_copy(acc_x.at[pl.ds(s * rpt, rpt)], zb.at[pl.ds(0, rpt)])
        pltpu.sync_copy(zb.at[pl.ds(0, rpt)],
                        out_hbm.at[pl.ds((c * 2 + 1) * n_pad + s * rpt,
                                         rpt)])

    return sc_count


def _embed(xf, emb_ref):
    return emb_ref[0:1, :] * (1.0 - xf) + emb_ref[1:2, :] * xf


def _bn(z, g, b):
    mu = jnp.mean(z, axis=0, keepdims=True)
    zc = z - mu
    var = jnp.mean(zc * zc, axis=0, keepdims=True)
    return zc * lax.rsqrt(var + 1e-5) * g + b


def _gin_tail(z, w1_ref, w2_ref, g1_ref, b1_ref, g2_ref, b2_ref, out_ref):
    z = jnp.dot(z, w1_ref[...], preferred_element_type=jnp.float32)
    z = _bn(z, g1_ref[...], b1_ref[...])
    z = jnp.maximum(z, 0.0)
    z = jnp.dot(z, w2_ref[...], preferred_element_type=jnp.float32)
    z = _bn(z, g2_ref[...], b2_ref[...])
    out_ref[...] = jnp.where(z > 0.0, z, jnp.exp(jnp.minimum(z, 0.0)) - 1.0)


def _layer_body(h_ref, agg_ref, w1_ref, w2_ref, g1_ref, b1_ref, g2_ref,
                b2_ref, out_ref):
    n = h_ref.shape[0]
    z = h_ref[...] + agg_ref[0, :n, :] + agg_ref[1, :n, :]
    _gin_tail(z, w1_ref, w2_ref, g1_ref, b1_ref, g2_ref, b2_ref, out_ref)


def _layer1_body(xf_ref, emb_ref, d0_ref, x0_ref, d1_ref, x1_ref, w1_ref,
                 w2_ref, g1_ref, b1_ref, g2_ref, b2_ref, out_ref):
    xf = xf_ref[...]
    e0 = emb_ref[0:1, :]
    e1 = emb_ref[1:2, :]
    h0 = e0 * (1.0 - xf) + e1 * xf
    deg = d0_ref[...] + d1_ref[...]
    s1 = x0_ref[...] + x1_ref[...]
    z = h0 + (deg - s1) * e0 + s1 * e1
    _gin_tail(z, w1_ref, w2_ref, g1_ref, b1_ref, g2_ref, b2_ref, out_ref)


def _readout_body(xf_ref, emb_ref, h1_ref, h2_ref, h3_ref, wr1_ref, br1_ref,
                  wr2_ref, br2_ref, out_ref):
    hdim = h1_ref.shape[1]
    h0 = _embed(xf_ref[...], emb_ref)
    acc = jnp.dot(h0, wr1_ref[0 * hdim:1 * hdim, :],
                  preferred_element_type=jnp.float32)
    acc += jnp.dot(h1_ref[...], wr1_ref[1 * hdim:2 * hdim, :],
                   preferred_element_type=jnp.float32)
    acc += jnp.dot(h2_ref[...], wr1_ref[2 * hdim:3 * hdim, :],
                   preferred_element_type=jnp.float32)
    acc += jnp.dot(h3_ref[...], wr1_ref[3 * hdim:4 * hdim, :],
                   preferred_element_type=jnp.float32)
    acc = jnp.maximum(acc + br1_ref[...], 0.0)
    out_ref[...] = jnp.dot(acc, wr2_ref[...],
                           preferred_element_type=jnp.float32) + br2_ref[...]


def _tc_call(body, out_shape, *args):
    return pl.pallas_call(
        body, out_shape=jax.ShapeDtypeStruct(out_shape, jnp.float32))(*args)


def kernel(x, edge_index, emb, W1, W2, bn1_g, bn1_b, bn2_g, bn2_b,
           Wr1, br1, Wr2, br2):
    n = x.shape[0]
    e = edge_index.shape[1]
    hdim = emb.shape[1]
    nlayers = W1.shape[0]

    chunk = 112
    T = -(-e // (NS * chunk))
    T = ((T + 5) // 6) * 6
    e_pad = T * NS * chunk
    nchunks_cnt = T // 2
    nch0 = max(6, min(T - 6, int(round(0.85 * T / 3.0)) * 3))
    nch1 = T - nch0
    n_pad = ((n + 8 * NS - 1) // (8 * NS)) * (8 * NS)
    if e_pad > e and n_pad == n:
        n_pad += 8 * NS

    src = edge_index[0].astype(jnp.int32)
    dst = edge_index[1].astype(jnp.int32)
    if e_pad > e:
        src = jnp.concatenate([src, jnp.zeros((e_pad - e,), jnp.int32)])
        dst = jnp.concatenate([dst, jnp.full((e_pad - e,), n, jnp.int32)])
    xf = x.astype(jnp.float32).reshape(n, 1)
    zero = jnp.zeros((n_pad, hdim), jnp.float32)
    xf1 = x.astype(jnp.float32)

    sc_scatter = _make_sc_scatter(n_pad, hdim, nch0, nch1, chunk)
    sc_count = _make_sc_count(n_pad, nchunks_cnt, chunk)

    def bn_args(i):
        return (bn1_g[i].reshape(1, hdim), bn1_b[i].reshape(1, hdim),
                bn2_g[i].reshape(1, hdim), bn2_b[i].reshape(1, hdim))

    cnt4 = sc_count(xf1, src, dst).reshape(4, n_pad)
    cnt_args = [cnt4[k, :n].reshape(n, 1) for k in range(4)]
    h = _tc_call(_layer1_body, (n, hdim), xf, emb, *cnt_args, W1[0], W2[0],
                 *bn_args(0))
    hs = [h]
    for i in range(1, nlayers):
        agg = sc_scatter(h, src, dst, zero)
        h = _tc_call(_layer_body, (n, hdim), h, agg, W1[i], W2[i],
                     *bn_args(i))
        hs.append(h)

    out = _tc_call(_readout_body, (n, 1), xf, emb, hs[0], hs[1], hs[2],
                   Wr1, br1.reshape(1, hdim), Wr2, br2.reshape(1, 1))
    return out


# speedup vs baseline: 2.3533x; 1.0197x over previous
import functools

import jax
import jax.numpy as jnp
from jax import lax
from jax.experimental import pallas as pl
from jax.experimental.pallas import tpu as pltpu
from jax.experimental.pallas import tpu_sc as plsc

NC = 2
NS = 16
NW = NC * NS


NB = 3


def _make_sc_scatter(n_pad, h, nch0, nch1, chunk):
    rpt = n_pad // NS

    mesh = plsc.VectorSubcoreMesh(core_axis_name="c", subcore_axis_name="s")

    @functools.partial(
        pl.kernel,
        out_type=jax.ShapeDtypeStruct((NC, n_pad, h), jnp.float32),
        mesh=mesh,
        scratch_types=[
            pltpu.VMEM((NB, chunk), jnp.int32),
            pltpu.VMEM((NB, chunk), jnp.int32),
            pltpu.VMEM((NB, chunk, h), jnp.float32),
            pltpu.VMEM_SHARED((n_pad, h), jnp.float32),
            [pltpu.SemaphoreType.DMA] * NB,
            [pltpu.SemaphoreType.DMA] * NB,
            [pltpu.SemaphoreType.DMA] * NB,
        ],
    )
    def sc_scatter(h_hbm, src_hbm, dst_hbm, zero_hbm, out_hbm,
                   srcv, dstv, rows, acc_sh, sem_i, sem_g, sem_s):
        c = lax.axis_index("c")
        s = lax.axis_index("s")

        pltpu.sync_copy(zero_hbm.at[pl.ds(s * rpt, rpt)],
                        acc_sh.at[pl.ds(s * rpt, rpt)])
        plsc.subcore_barrier()

        def run_pipe(nchunks, base):
            def stage_i(j, b):
                off = pl.multiple_of(base + j * chunk, 8)
                pltpu.async_copy(src_hbm.at[pl.ds(off, chunk)], srcv.at[b],
                                 sem_i[b])
                pltpu.async_copy(dst_hbm.at[pl.ds(off, chunk)], dstv.at[b],
                                 sem_i[b])

            def stage_a(j, b):
                off = pl.multiple_of(base + j * chunk, 8)
                pltpu.make_async_copy(src_hbm.at[pl.ds(off, chunk)],
                                      srcv.at[b], sem_i[b]).wait()
                pltpu.make_async_copy(dst_hbm.at[pl.ds(off, chunk)],
                                      dstv.at[b], sem_i[b]).wait()
                pltpu.async_copy(h_hbm.at[srcv.at[b]], rows.at[b], sem_g[b])

            def stage_b(j, b):
                pltpu.make_async_copy(h_hbm.at[srcv.at[b]], rows.at[b],
                                      sem_g[b]).wait()
                pltpu.async_copy(rows.at[b], acc_sh.at[dstv.at[b]], sem_s[b],
                                 add=True)

            def stage_c(j, b):
                pltpu.make_async_copy(rows.at[b], acc_sh.at[dstv.at[b]],
                                      sem_s[b]).wait()

            stage_i(0, 0)
            stage_i(1, 1)
            stage_a(0, 0)
            stage_i(2, 2)
            stage_a(1, 1)
            stage_b(0, 0)

            @pl.loop(1, nchunks - 2, step=NB)
            def _steady(i2):
                for bb in range(NB):
                    j = i2 + bb
                    stage_c(j - 1, bb)
                    stage_i(j + 2, bb)
                    stage_a(j + 1, (bb + 2) % NB)
                    stage_b(j, (bb + 1) % NB)

            stage_c(nchunks - 3, (nchunks - 3) % NB)
            stage_a(nchunks - 1, (nchunks - 1) % NB)
            stage_b(nchunks - 2, (nchunks - 2) % NB)
            stage_c(nchunks - 2, (nchunks - 2) % NB)
            stage_b(nchunks - 1, (nchunks - 1) % NB)
            stage_c(nchunks - 1, (nchunks - 1) % NB)

        @pl.when(c == 0)
        def _core0():
            run_pipe(nch0, s * nch0 * chunk)

        @pl.when(c == 1)
        def _core1():
            run_pipe(nch1, (NS * nch0 + s * nch1) * chunk)

        plsc.subcore_barrier()
        pltpu.sync_copy(acc_sh.at[pl.ds(s * rpt, rpt)],
                        out_hbm.at[c, pl.ds(s * rpt, rpt)])

    return sc_scatter


def _make_sc_count(n_pad, nchunks, chunk):
    rpt = n_pad // NS
    rptz = ((rpt + 15) // 16) * 16
    epw = nchunks * chunk

    mesh = plsc.VectorSubcoreMesh(core_axis_name="c", subcore_axis_name="s")

    @functools.partial(
        pl.kernel,
        out_type=jax.ShapeDtypeStruct((NC * 2 * n_pad,), jnp.float32),
        mesh=mesh,
        scratch_types=[
            pltpu.VMEM((NB, chunk), jnp.int32),
            pltpu.VMEM((NB, chunk), jnp.int32),
            pltpu.VMEM((NB, chunk), jnp.float32),
            pltpu.VMEM((chunk,), jnp.float32),
            pltpu.VMEM((rptz,), jnp.float32),
            pltpu.VMEM_SHARED((n_pad,), jnp.float32),
            pltpu.VMEM_SHARED((n_pad,), jnp.float32),
            [pltpu.SemaphoreType.DMA] * NB,
            [pltpu.SemaphoreType.DMA] * NB,
            [pltpu.SemaphoreType.DMA] * NB,
        ],
    )
    def sc_count(xf_hbm, src_hbm, dst_hbm, out_hbm,
                 srcv, dstv, vals, ones, zb, acc_d, acc_x,
                 sem_i, sem_g, sem_s):
        c = lax.axis_index("c")
        s = lax.axis_index("s")
        wid = s * NC + c
        base = wid * epw

        for k in range(chunk // 16):
            ones[pl.ds(k * 16, 16)] = jnp.full((16,), 1.0, jnp.float32)
        for k in range(rptz // 16):
            zb[pl.ds(k * 16, 16)] = jnp.zeros((16,), jnp.float32)
        pltpu.sync_copy(zb.at[pl.ds(0, rpt)], acc_d.at[pl.ds(s * rpt, rpt)])
        pltpu.sync_copy(zb.at[pl.ds(0, rpt)], acc_x.at[pl.ds(s * rpt, rpt)])
        plsc.subcore_barrier()

        def stage_i(j, b):
            off = pl.multiple_of(base + j * chunk, 8)
            pltpu.async_copy(src_hbm.at[pl.ds(off, chunk)], srcv.at[b],
                             sem_i[b])
            pltpu.async_copy(dst_hbm.at[pl.ds(off, chunk)], dstv.at[b],
                             sem_i[b])

        def stage_a(j, b):
            off = pl.multiple_of(base + j * chunk, 8)
            pltpu.make_async_copy(src_hbm.at[pl.ds(off, chunk)], srcv.at[b],
                                  sem_i[b]).wait()
            pltpu.make_async_copy(dst_hbm.at[pl.ds(off, chunk)], dstv.at[b],
                                  sem_i[b]).wait()
            pltpu.async_copy(xf_hbm.at[srcv.at[b]], vals.at[b], sem_g[b])

        def stage_b(j, b):
            pltpu.make_async_copy(xf_hbm.at[srcv.at[b]], vals.at[b],
                                  sem_g[b]).wait()
            pltpu.async_copy(vals.at[b], acc_x.at[dstv.at[b]], sem_s[b],
                             add=True)
            pltpu.async_copy(ones, acc_d.at[dstv.at[b]], sem_s[b], add=True)

        def stage_c(j, b):
            pltpu.make_async_copy(vals.at[b], acc_x.at[dstv.at[b]],
                                  sem_s[b]).wait()
            pltpu.make_async_copy(ones, acc_d.at[dstv.at[b]],
                                  sem_s[b]).wait()

        stage_i(0, 0)
        stage_i(1, 1)
        stage_a(0, 0)
        stage_i(2, 2)
        stage_a(1, 1)
        stage_b(0, 0)

        @pl.loop(1, nchunks - 2, step=NB)
        def _steady(i2):
            for bb in range(NB):
                j = i2 + bb
                stage_c(j - 1, bb)
                stage_i(j + 2, bb)
                stage_a(j + 1, (bb + 2) % NB)
                stage_b(j, (bb + 1) % NB)

        stage_c(nchunks - 3, (nchunks - 3) % NB)
        stage_a(nchunks - 1, (nchunks - 1) % NB)
        stage_b(nchunks - 2, (nchunks - 2) % NB)
        stage_c(nchunks - 2, (nchunks - 2) % NB)
        stage_b(nchunks - 1, (nchunks - 1) % NB)
        stage_c(nchunks - 1, (nchunks - 1) % NB)

        plsc.subcore_barrier()
        pltpu.sync_copy(acc_d.at[pl.ds(s * rpt, rpt)], zb.at[pl.ds(0, rpt)])
        pltpu.sync_copy(zb.at[pl.ds(0, rpt)],
                        out_hbm.at[pl.ds(c * 2 * n_pad + s * rpt, rpt)])
        pltpu.sync_copy(acc_x.at[pl.ds(s * rpt, rpt)], zb.at[pl.ds(0, rpt)])
        pltpu.sync_copy(zb.at[pl.ds(0, rpt)],
                        out_hbm.at[pl.ds((c * 2 + 1) * n_pad + s * rpt,
                                         rpt)])

    return sc_count


def _embed(xf, emb_ref):
    return emb_ref[0:1, :] * (1.0 - xf) + emb_ref[1:2, :] * xf


def _bn(z, g, b):
    mu = jnp.mean(z, axis=0, keepdims=True)
    zc = z - mu
    var = jnp.mean(zc * zc, axis=0, keepdims=True)
    return zc * lax.rsqrt(var + 1e-5) * g + b


def _gin_block(z, w1_ref, w2_ref, g1_ref, b1_ref, g2_ref, b2_ref):
    z = jnp.dot(z, w1_ref[...], preferred_element_type=jnp.float32)
    z = _bn(z, g1_ref[...], b1_ref[...])
    z = jnp.maximum(z, 0.0)
    z = jnp.dot(z, w2_ref[...], preferred_element_type=jnp.float32)
    z = _bn(z, g2_ref[...], b2_ref[...])
    return jnp.where(z > 0.0, z, jnp.exp(jnp.minimum(z, 0.0)) - 1.0)


def _gin_tail(z, w1_ref, w2_ref, g1_ref, b1_ref, g2_ref, b2_ref, out_ref):
    out_ref[...] = _gin_block(z, w1_ref, w2_ref, g1_ref, b1_ref, g2_ref,
                              b2_ref)


def _layer_body(h_ref, agg_ref, w1_ref, w2_ref, g1_ref, b1_ref, g2_ref,
                b2_ref, out_ref):
    n = h_ref.shape[0]
    z = h_ref[...] + agg_ref[0, :n, :] + agg_ref[1, :n, :]
    _gin_tail(z, w1_ref, w2_ref, g1_ref, b1_ref, g2_ref, b2_ref, out_ref)


def _layer1_body(xf_ref, emb_ref, d0_ref, x0_ref, d1_ref, x1_ref, w1_ref,
                 w2_ref, g1_ref, b1_ref, g2_ref, b2_ref, out_ref):
    xf = xf_ref[...]
    e0 = emb_ref[0:1, :]
    e1 = emb_ref[1:2, :]
    h0 = e0 * (1.0 - xf) + e1 * xf
    deg = d0_ref[...] + d1_ref[...]
    s1 = x0_ref[...] + x1_ref[...]
    z = h0 + (deg - s1) * e0 + s1 * e1
    _gin_tail(z, w1_ref, w2_ref, g1_ref, b1_ref, g2_ref, b2_ref, out_ref)


def _last_layer_readout_body(h2_ref, agg_ref, w1_ref, w2_ref, g1_ref, b1_ref,
                             g2_ref, b2_ref, xf_ref, emb_ref, h1_ref,
                             wr1_ref, br1_ref, wr2_ref, br2_ref, out_ref):
    n = h2_ref.shape[0]
    hdim = h2_ref.shape[1]
    z = h2_ref[...] + agg_ref[0, :n, :] + agg_ref[1, :n, :]
    h3 = _gin_block(z, w1_ref, w2_ref, g1_ref, b1_ref, g2_ref, b2_ref)
    h0 = _embed(xf_ref[...], emb_ref)
    acc = jnp.dot(h0, wr1_ref[0 * hdim:1 * hdim, :],
                  preferred_element_type=jnp.float32)
    acc += jnp.dot(h1_ref[...], wr1_ref[1 * hdim:2 * hdim, :],
                   preferred_element_type=jnp.float32)
    acc += jnp.dot(h2_ref[...], wr1_ref[2 * hdim:3 * hdim, :],
                   preferred_element_type=jnp.float32)
    acc += jnp.dot(h3, wr1_ref[3 * hdim:4 * hdim, :],
                   preferred_element_type=jnp.float32)
    acc = jnp.maximum(acc + br1_ref[...], 0.0)
    out_ref[...] = jnp.dot(acc, wr2_ref[...],
                           preferred_element_type=jnp.float32) + br2_ref[...]


def _tc_call(body, out_shape, *args):
    return pl.pallas_call(
        body, out_shape=jax.ShapeDtypeStruct(out_shape, jnp.float32))(*args)


def kernel(x, edge_index, emb, W1, W2, bn1_g, bn1_b, bn2_g, bn2_b,
           Wr1, br1, Wr2, br2):
    n = x.shape[0]
    e = edge_index.shape[1]
    hdim = emb.shape[1]
    nlayers = W1.shape[0]

    chunk = 112
    T = -(-e // (NS * chunk))
    T = ((T + 5) // 6) * 6
    e_pad = T * NS * chunk
    nchunks_cnt = T // 2
    nch0 = max(6, min(T - 6, int(round(0.85 * T / 3.0)) * 3))
    nch1 = T - nch0
    n_pad = ((n + 8 * NS - 1) // (8 * NS)) * (8 * NS)
    if e_pad > e and n_pad == n:
        n_pad += 8 * NS

    src = edge_index[0].astype(jnp.int32)
    dst = edge_index[1].astype(jnp.int32)
    if e_pad > e:
        src = jnp.concatenate([src, jnp.zeros((e_pad - e,), jnp.int32)])
        dst = jnp.concatenate([dst, jnp.full((e_pad - e,), n, jnp.int32)])
    xf = x.astype(jnp.float32).reshape(n, 1)
    zero = jnp.zeros((n_pad, hdim), jnp.float32)
    xf1 = x.astype(jnp.float32)

    sc_scatter = _make_sc_scatter(n_pad, hdim, nch0, nch1, chunk)
    sc_count = _make_sc_count(n_pad, nchunks_cnt, chunk)

    def bn_args(i):
        return (bn1_g[i].reshape(1, hdim), bn1_b[i].reshape(1, hdim),
                bn2_g[i].reshape(1, hdim), bn2_b[i].reshape(1, hdim))

    cnt4 = sc_count(xf1, src, dst).reshape(4, n_pad)
    cnt_args = [cnt4[k, :n].reshape(n, 1) for k in range(4)]
    h = _tc_call(_layer1_body, (n, hdim), xf, emb, *cnt_args, W1[0], W2[0],
                 *bn_args(0))
    hs = [h]
    for i in range(1, nlayers - 1):
        agg = sc_scatter(h, src, dst, zero)
        h = _tc_call(_layer_body, (n, hdim), h, agg, W1[i], W2[i],
                     *bn_args(i))
        hs.append(h)

    agg = sc_scatter(h, src, dst, zero)
    out = _tc_call(_last_layer_readout_body, (n, 1), hs[1], agg,
                   W1[nlayers - 1], W2[nlayers - 1], *bn_args(nlayers - 1),
                   xf, emb, hs[0], Wr1, br1.reshape(1, hdim), Wr2,
                   br2.reshape(1, 1))
    return out
